# baseline (speedup 1.0000x reference)
import functools

import jax
import jax.numpy as jnp
from jax import lax
from jax.experimental import pallas as pl
from jax.experimental.pallas import tpu as pltpu
from jax.experimental.pallas import tpu_sc as plsc

N = 16384
EMB_DIM = 64
CHG_DIM = 8
OUT_DIM = EMB_DIM + CHG_DIM

_info = plsc.get_sparse_core_info()
NC, NS = _info.num_cores, _info.num_subcores
NW = NC * NS
B_PER_W = N // NW
CHUNK = 128
N_CHUNKS = B_PER_W // CHUNK

_mesh = plsc.VectorSubcoreMesh(core_axis_name="c", subcore_axis_name="s")


@functools.partial(
    pl.kernel,
    mesh=_mesh,
    out_type=jax.ShapeDtypeStruct((N, OUT_DIM), jnp.float32),
    scratch_types=[
        pltpu.VMEM((N_CHUNKS, CHUNK), jnp.int32),
        pltpu.VMEM((B_PER_W, EMB_DIM), jnp.float32),
        pltpu.VMEM((B_PER_W, CHG_DIM), jnp.float32),
        pltpu.SemaphoreType.DMA,
    ],
    compiler_params=pltpu.CompilerParams(use_tc_tiling_on_sc=False),
)
def _emb_kernel(idx_hbm, charge_hbm, table_hbm, out_hbm,
                idx_v, rows_v, chg_v, sem):
    wid = lax.axis_index("s") * NC + lax.axis_index("c")
    base = wid * B_PER_W

    pltpu.sync_copy(idx_hbm.at[wid], idx_v)

    copies = []
    for j in range(N_CHUNKS):
        copies.append(pltpu.async_copy(
            table_hbm.at[idx_v.at[j]],
            rows_v.at[pl.ds(j * CHUNK, CHUNK)],
            sem,
        ))
    pltpu.sync_copy(charge_hbm.at[pl.ds(base, B_PER_W)], chg_v)
    for c in copies:
        c.wait()

    pltpu.sync_copy(rows_v, out_hbm.at[pl.ds(base, B_PER_W), pl.ds(0, EMB_DIM)])
    pltpu.sync_copy(chg_v, out_hbm.at[pl.ds(base, B_PER_W), pl.ds(EMB_DIM, CHG_DIM)])


def kernel(atom_types, charge, pos, emb_table):
    idx = atom_types.reshape(NW, N_CHUNKS, CHUNK).astype(jnp.int32)
    out = _emb_kernel(idx, charge, emb_table)
    return out.astype(pos.dtype)
